# baseline (speedup 1.0000x reference)
import jax
import jax.numpy as jnp
import numpy as np
from jax import lax
from jax.experimental import pallas as pl
from jax.experimental.pallas import tpu as pltpu
from jax.experimental.pallas import tpu_sc as plsc

B = 4096
SEQ = 200
D = 64
EMB_DIM = 64
N_EMB = 1000000
N_OUT = 1
NC = 2
NS = 16
NW = NC * NS
RPW = B // NW
S0 = 104
S1 = SEQ - S0
NBUF = 4
PAD = 256

_PERM = np.concatenate([np.arange(0, 32, 2), np.arange(1, 32, 2),
                        np.arange(32, 64, 2), np.arange(33, 64, 2)])


def _pool_body(x_hbm, tbl_hbm, out_hbm, idx_v, ring_v, acc_v, *sems):
    cid = lax.axis_index("c")
    sid = lax.axis_index("s")
    wid = sid * NC + cid

    pltpu.sync_copy(x_hbm.at[pl.ds(wid * (RPW * PAD), RPW * PAD)], idx_v)

    def remap(k, carry):
        v = idx_v[pl.ds(k * 16, 16)]
        idx_v[pl.ds(k * 16, 16)] = jnp.where(v >= H, 2 * v - (2 * H - 1),
                                             2 * v)
        return carry

    lax.fori_loop(0, RPW * PAD // 16, remap, 0)

    def fire(b, row):
        pltpu.async_copy(tbl_hbm.at[idx_v.at[pl.ds(row * PAD, S0)]],
                         ring_v.at[b, pl.ds(0, S0)], sems[b])
        pltpu.async_copy(tbl_hbm.at[idx_v.at[pl.ds(row * PAD + S0, S1)]],
                         ring_v.at[b, pl.ds(S0, S1)], sems[b])

    for b in range(NBUF):
        fire(b, b)

    def outer(g, carry):
        for b in range(NBUF):
            r = g * NBUF + b
            pltpu.make_async_copy(tbl_hbm.at[pl.ds(0, SEQ)],
                                  ring_v.at[b], sems[b]).wait()

            def red(j, acc):
                a0, a1, a2, a3 = acc
                v = ring_v[b, j, pl.ds(0, 16)]
                w = ring_v[b, j, pl.ds(16, 16)]
                hi = jnp.full((16,), -65536, jnp.int32)
                bc = jax.lax.bitcast_convert_type
                u0 = bc(v << 16, jnp.float32)
                u1 = bc(v & hi, jnp.float32)
                u2 = bc(w << 16, jnp.float32)
                u3 = bc(w & hi, jnp.float32)
                return (a0 + u0, a1 + u1, a2 + u2, a3 + u3)

            z = jnp.zeros((16,), jnp.float32)
            a0, a1, a2, a3 = lax.fori_loop(0, SEQ, red, (z, z, z, z))
            acc_v[r, pl.ds(0, 16)] = a0
            acc_v[r, pl.ds(16, 16)] = a1
            acc_v[r, pl.ds(32, 16)] = a2
            acc_v[r, pl.ds(48, 16)] = a3

            nxt = r + NBUF

            @pl.when(nxt < RPW)
            def _refire():
                fire(b, nxt)
        return carry

    lax.fori_loop(0, RPW // NBUF, outer, 0)
    pltpu.sync_copy(acc_v, out_hbm.at[pl.ds(wid * RPW, RPW)])


_POOL = pl.kernel(
    _pool_body,
    out_type=jax.ShapeDtypeStruct((B, D), jnp.float32),
    mesh=plsc.VectorSubcoreMesh(core_axis_name="c", subcore_axis_name="s"),
    scratch_types=(
        [pltpu.VMEM((RPW * PAD,), jnp.int32),
         pltpu.VMEM((NBUF, SEQ, D // 2), jnp.int32),
         pltpu.VMEM((RPW, D), jnp.float32)]
        + [pltpu.SemaphoreType.DMA] * NBUF
    ),
    compiler_params=pltpu.CompilerParams(use_tc_tiling_on_sc=False),
)


CT = 4096
H = 124 * CT
NPAD = 2 * H


def _tr_body(ta_ref, tb_ref, o_ref):
    c = jnp.concatenate([ta_ref[...], tb_ref[...]], axis=0)
    eye = jnp.eye(2 * EMB_DIM, dtype=jnp.float32)
    dn = (((0,), (0,)), ((), ()))
    t = lax.dot_general(c, eye, dn, preferred_element_type=jnp.float32)
    o_ref[...] = t.astype(jnp.bfloat16)


_TR = pl.pallas_call(
    _tr_body,
    grid=(H // CT,),
    in_specs=[pl.BlockSpec((EMB_DIM, CT), lambda g: (0, g)),
              pl.BlockSpec((EMB_DIM, CT),
                           lambda g: (0, jnp.minimum(g + H // CT,
                                                     (N_EMB - 1) // CT)))],
    out_specs=pl.BlockSpec((CT, 128), lambda g: (g, 0)),
    out_shape=jax.ShapeDtypeStruct((H, 128), jnp.bfloat16),
)


def _flat_body(x_ref, o_ref):
    v = x_ref[...]
    z = jnp.zeros((B, PAD - SEQ), jnp.int32)
    o_ref[...] = jnp.concatenate([v, z], axis=1).reshape(B * PAD)


_FLAT = pl.pallas_call(
    _flat_body,
    out_shape=jax.ShapeDtypeStruct((B * PAD,), jnp.int32),
)


def _mlp_body(s_ref, w1_ref, b1_ref, w2_ref, b2_ref, wo_ref, bo_ref, o_ref):
    h = s_ref[...] * (1.0 / SEQ)
    h = jnp.maximum(
        jnp.dot(h, w1_ref[...], preferred_element_type=jnp.float32)
        + b1_ref[...], 0.0)
    h = jnp.maximum(
        jnp.dot(h, w2_ref[...], preferred_element_type=jnp.float32)
        + b2_ref[...], 0.0)
    o_ref[...] = (
        jnp.dot(h, wo_ref[...], preferred_element_type=jnp.float32)
        + bo_ref[...])


_MLP = pl.pallas_call(
    _mlp_body,
    out_shape=jax.ShapeDtypeStruct((B, N_OUT), jnp.float32),
)


def kernel(x, emb_table, W1, b1, W2, b2, W_out, b_out):
    x1 = _FLAT(x.astype(jnp.int32))
    tbl_t = emb_table.T
    tbl = jax.lax.bitcast_convert_type(
        _TR(tbl_t, tbl_t).reshape(H, 64, 2), jnp.int32)
    tbl = tbl.reshape(-1).reshape(NPAD, D // 2)
    sums = _POOL(x1, tbl)
    W1p = W1[_PERM, :]
    return _MLP(sums, W1p, b1.reshape(1, D), W2, b2.reshape(1, D),
                W_out, b_out.reshape(1, N_OUT))


# speedup vs baseline: 4.8489x; 4.8489x over previous
import jax
import jax.numpy as jnp
import numpy as np
from jax import lax
from jax.experimental import pallas as pl
from jax.experimental.pallas import tpu as pltpu
from jax.experimental.pallas import tpu_sc as plsc

B = 4096
SEQ = 200
D = 64
EMB_DIM = 64
N_EMB = 1000000
N_OUT = 1
NC = 2
NS = 16
NW = NC * NS
RPW = B // NW
S0 = 104
S1 = SEQ - S0
NBUF = 4
PAD = 256

_PERM = np.concatenate([np.arange(0, 16), np.arange(32, 48),
                        np.arange(16, 32), np.arange(48, 64)])


def _pool_body(x_hbm, tbl_hbm, out_hbm, idx_v, ring_v, acc_v, *sems):
    cid = lax.axis_index("c")
    sid = lax.axis_index("s")
    wid = sid * NC + cid

    pltpu.sync_copy(x_hbm.at[pl.ds(wid * (RPW * PAD), RPW * PAD)], idx_v)

    def remap(k, carry):
        v = idx_v[pl.ds(k * 16, 16)]
        kk = jnp.int32(4 * Q4 - 1)
        z = jnp.int32(0)
        adj = (jnp.where(v >= Q4, kk, z)
               + jnp.where(v >= 2 * Q4, kk, z)
               + jnp.where(v >= 3 * Q4, kk, z))
        idx_v[pl.ds(k * 16, 16)] = 4 * v - adj
        return carry

    lax.fori_loop(0, RPW * PAD // 16, remap, 0)

    def fire(b, row):
        pltpu.async_copy(tbl_hbm.at[idx_v.at[pl.ds(row * PAD, S0)]],
                         ring_v.at[b, pl.ds(0, S0)], sems[b])
        pltpu.async_copy(tbl_hbm.at[idx_v.at[pl.ds(row * PAD + S0, S1)]],
                         ring_v.at[b, pl.ds(S0, S1)], sems[b])

    for b in range(NBUF):
        fire(b, b)

    def outer(g, carry):
        for b in range(NBUF):
            r = g * NBUF + b
            pltpu.make_async_copy(tbl_hbm.at[pl.ds(0, SEQ)],
                                  ring_v.at[b], sems[b]).wait()

            def red(j, acc):
                a0, a1, a2, a3 = acc
                v = ring_v[b, j, pl.ds(0, 16)]
                w = ring_v[b, j, pl.ds(16, 16)]
                hi = jnp.full((16,), -65536, jnp.int32)
                bc = jax.lax.bitcast_convert_type
                u0 = bc(v << 16, jnp.float32)
                u1 = bc(v & hi, jnp.float32)
                u2 = bc(w << 16, jnp.float32)
                u3 = bc(w & hi, jnp.float32)
                return (a0 + u0, a1 + u1, a2 + u2, a3 + u3)

            z = jnp.zeros((16,), jnp.float32)
            a0, a1, a2, a3 = lax.fori_loop(0, SEQ, red, (z, z, z, z))
            acc_v[r, pl.ds(0, 16)] = a0
            acc_v[r, pl.ds(16, 16)] = a1
            acc_v[r, pl.ds(32, 16)] = a2
            acc_v[r, pl.ds(48, 16)] = a3

            nxt = r + NBUF

            @pl.when(nxt < RPW)
            def _refire():
                fire(b, nxt)
        return carry

    lax.fori_loop(0, RPW // NBUF, outer, 0)
    pltpu.sync_copy(acc_v, out_hbm.at[pl.ds(wid * RPW, RPW)])


_POOL = pl.kernel(
    _pool_body,
    out_type=jax.ShapeDtypeStruct((B, D), jnp.float32),
    mesh=plsc.VectorSubcoreMesh(core_axis_name="c", subcore_axis_name="s"),
    scratch_types=(
        [pltpu.VMEM((RPW * PAD,), jnp.int32),
         pltpu.VMEM((NBUF, SEQ, D // 2), jnp.int32),
         pltpu.VMEM((RPW, D), jnp.float32)]
        + [pltpu.SemaphoreType.DMA] * NBUF
    ),
    compiler_params=pltpu.CompilerParams(use_tc_tiling_on_sc=False),
)


CT = 2048
Q4 = 124 * CT
NPAD = 4 * Q4


def _tr_body(t0_ref, t1_ref, t2_ref, t3_ref, o_ref):
    c = jnp.concatenate([t0_ref[...], t1_ref[...],
                         t2_ref[...], t3_ref[...]], axis=0)
    eye = jnp.eye(4 * EMB_DIM, dtype=jnp.bfloat16)
    dn = (((0,), (0,)), ((), ()))
    t = lax.dot_general(c.astype(jnp.bfloat16), eye, dn,
                        preferred_element_type=jnp.float32)
    bits = jax.lax.bitcast_convert_type(
        t.astype(jnp.bfloat16), jnp.uint16).astype(jnp.uint32)
    ws = []
    for q in range(4):
        lo = bits[:, q * 64:q * 64 + 32]
        hi = bits[:, q * 64 + 32:q * 64 + 64]
        ws.append(lo | (hi << 16))
    o_ref[...] = jax.lax.bitcast_convert_type(
        jnp.concatenate(ws, axis=1), jnp.int32)


def _mk_in_spec(q):
    return pl.BlockSpec(
        (EMB_DIM, CT),
        lambda g: (0, jnp.minimum(g + q * (Q4 // CT), (N_EMB - 1) // CT)))


_TR = pl.pallas_call(
    _tr_body,
    grid=(Q4 // CT,),
    in_specs=[_mk_in_spec(q) for q in range(4)],
    out_specs=pl.BlockSpec((CT, 128), lambda g: (g, 0)),
    out_shape=jax.ShapeDtypeStruct((Q4, 128), jnp.int32),
)


def _flat_body(x_ref, o_ref):
    v = x_ref[...]
    z = jnp.zeros((B, PAD - SEQ), jnp.int32)
    o_ref[...] = jnp.concatenate([v, z], axis=1).reshape(B * PAD)


_FLAT = pl.pallas_call(
    _flat_body,
    out_shape=jax.ShapeDtypeStruct((B * PAD,), jnp.int32),
)


def _mlp_body(s_ref, w1_ref, b1_ref, w2_ref, b2_ref, wo_ref, bo_ref, o_ref):
    h = s_ref[...] * (1.0 / SEQ)
    h = jnp.maximum(
        jnp.dot(h, w1_ref[...], preferred_element_type=jnp.float32)
        + b1_ref[...], 0.0)
    h = jnp.maximum(
        jnp.dot(h, w2_ref[...], preferred_element_type=jnp.float32)
        + b2_ref[...], 0.0)
    o_ref[...] = (
        jnp.dot(h, wo_ref[...], preferred_element_type=jnp.float32)
        + bo_ref[...])


_MLP = pl.pallas_call(
    _mlp_body,
    out_shape=jax.ShapeDtypeStruct((B, N_OUT), jnp.float32),
)


def kernel(x, emb_table, W1, b1, W2, b2, W_out, b_out):
    x1 = _FLAT(x.astype(jnp.int32))
    tbl_t = emb_table.T
    tbl = _TR(tbl_t, tbl_t, tbl_t, tbl_t).reshape(-1).reshape(NPAD, D // 2)
    sums = _POOL(x1, tbl)
    W1p = W1[_PERM, :]
    return _MLP(sums, W1p, b1.reshape(1, D), W2, b2.reshape(1, D),
                W_out, b_out.reshape(1, N_OUT))


# speedup vs baseline: 5.6962x; 1.1747x over previous
import jax
import jax.numpy as jnp
from jax import lax
from jax.experimental import pallas as pl
from jax.experimental.pallas import tpu as pltpu
from jax.experimental.pallas import tpu_sc as plsc

B = 4096
SEQ = 200
D = 64
EMB_DIM = 64
N_EMB = 1000000
N_OUT = 1
NC = 2
NS = 16
NW = NC * NS
RPW = B // NW
S0 = 104
S1 = SEQ - S0
NBUF = 4
PAD = 256


def _pool_body(x_hbm, tbl_hbm, out_hbm, idx_v, ring_v, acc_v, *sems):
    cid = lax.axis_index("c")
    sid = lax.axis_index("s")
    wid = sid * NC + cid

    pltpu.sync_copy(x_hbm.at[pl.ds(wid * (RPW * PAD), RPW * PAD)], idx_v)

    def remap(k, carry):
        v = idx_v[pl.ds(k * 16, 16)]
        idx_v[pl.ds(k * 16, 16)] = jnp.where(v >= H, 2 * v - (2 * H - 1),
                                             2 * v)
        return carry

    lax.fori_loop(0, RPW * PAD // 16, remap, 0)

    def fire(b, row):
        pltpu.async_copy(tbl_hbm.at[idx_v.at[pl.ds(row * PAD, S0)]],
                         ring_v.at[b, pl.ds(0, S0)], sems[b])
        pltpu.async_copy(tbl_hbm.at[idx_v.at[pl.ds(row * PAD + S0, S1)]],
                         ring_v.at[b, pl.ds(S0, S1)], sems[b])

    for b in range(NBUF):
        fire(b, b)

    def outer(g, carry):
        for b in range(NBUF):
            r = g * NBUF + b
            pltpu.make_async_copy(tbl_hbm.at[pl.ds(0, SEQ)],
                                  ring_v.at[b], sems[b]).wait()

            def red(j, acc):
                a0, a1, a2, a3 = acc
                a0 = a0 + ring_v[b, j, pl.ds(0, 16)]
                a1 = a1 + ring_v[b, j, pl.ds(16, 16)]
                a2 = a2 + ring_v[b, j, pl.ds(32, 16)]
                a3 = a3 + ring_v[b, j, pl.ds(48, 16)]
                return (a0, a1, a2, a3)

            z = jnp.zeros((16,), jnp.float32)
            a0, a1, a2, a3 = lax.fori_loop(0, SEQ, red, (z, z, z, z))
            acc_v[r, pl.ds(0, 16)] = a0
            acc_v[r, pl.ds(16, 16)] = a1
            acc_v[r, pl.ds(32, 16)] = a2
            acc_v[r, pl.ds(48, 16)] = a3

            nxt = r + NBUF

            @pl.when(nxt < RPW)
            def _refire():
                fire(b, nxt)
        return carry

    lax.fori_loop(0, RPW // NBUF, outer, 0)
    pltpu.sync_copy(acc_v, out_hbm.at[pl.ds(wid * RPW, RPW)])


_POOL = pl.kernel(
    _pool_body,
    out_type=jax.ShapeDtypeStruct((B, D), jnp.float32),
    mesh=plsc.VectorSubcoreMesh(core_axis_name="c", subcore_axis_name="s"),
    scratch_types=(
        [pltpu.VMEM((RPW * PAD,), jnp.int32),
         pltpu.VMEM((NBUF, SEQ, D), jnp.float32),
         pltpu.VMEM((RPW, D), jnp.float32)]
        + [pltpu.SemaphoreType.DMA] * NBUF
    ),
    compiler_params=pltpu.CompilerParams(use_tc_tiling_on_sc=False),
)


CT = 4096
H = 124 * CT
NPAD = 2 * H


def _tr_body(ta_ref, tb_ref, o_ref):
    c = jnp.concatenate([ta_ref[...], tb_ref[...]], axis=0)
    eye = jnp.eye(2 * EMB_DIM, dtype=jnp.float32)
    dn = (((0,), (0,)), ((), ()))
    o_ref[...] = lax.dot_general(c, eye, dn,
                                 preferred_element_type=jnp.float32)


_TR = pl.pallas_call(
    _tr_body,
    grid=(H // CT,),
    in_specs=[pl.BlockSpec((EMB_DIM, CT), lambda g: (0, g)),
              pl.BlockSpec((EMB_DIM, CT),
                           lambda g: (0, jnp.minimum(g + H // CT,
                                                     (N_EMB - 1) // CT)))],
    out_specs=pl.BlockSpec((CT, 128), lambda g: (g, 0)),
    out_shape=jax.ShapeDtypeStruct((H, 128), jnp.float32),
)


def _flat_body(x_ref, o_ref):
    v = x_ref[...]
    z = jnp.zeros((B, PAD - SEQ), jnp.int32)
    o_ref[...] = jnp.concatenate([v, z], axis=1).reshape(B * PAD)


_FLAT = pl.pallas_call(
    _flat_body,
    out_shape=jax.ShapeDtypeStruct((B * PAD,), jnp.int32),
)


def _mlp_body(s_ref, w1_ref, b1_ref, w2_ref, b2_ref, wo_ref, bo_ref, o_ref):
    h = s_ref[...] * (1.0 / SEQ)
    h = jnp.maximum(
        jnp.dot(h, w1_ref[...], preferred_element_type=jnp.float32)
        + b1_ref[...], 0.0)
    h = jnp.maximum(
        jnp.dot(h, w2_ref[...], preferred_element_type=jnp.float32)
        + b2_ref[...], 0.0)
    o_ref[...] = (
        jnp.dot(h, wo_ref[...], preferred_element_type=jnp.float32)
        + bo_ref[...])


_MLP = pl.pallas_call(
    _mlp_body,
    out_shape=jax.ShapeDtypeStruct((B, N_OUT), jnp.float32),
)


def kernel(x, emb_table, W1, b1, W2, b2, W_out, b_out):
    x1 = _FLAT(x.astype(jnp.int32))
    tbl_t = emb_table.T
    tbl = _TR(tbl_t, tbl_t).reshape(-1).reshape(NPAD, D)
    sums = _POOL(x1, tbl)
    return _MLP(sums, W1, b1.reshape(1, D), W2, b2.reshape(1, D),
                W_out, b_out.reshape(1, N_OUT))
